# Initial kernel scaffold; baseline (speedup 1.0000x reference)
import jax
import jax.numpy as jnp
from jax.experimental import pallas as pl


def kernel(x, y, triplet_flag, debug, emb_table, bias, W, b):
    raise NotImplementedError("write your pallas kernel here")


# speedup vs baseline: 1.9772x; 1.9772x over previous
import functools

import jax
import jax.numpy as jnp
from jax import lax
from jax.experimental import pallas as pl
from jax.experimental.pallas import tpu as pltpu
from jax.experimental.pallas import tpu_sc as plsc

B = 16384
H = 200
D = 128
OUT = 1000

NC = 2
NS = 16
NW = NC * NS
B_PER_W = B // NW
HC = H // 2
ACC = 32


def _emb_sum(x3, table):
  mesh = plsc.VectorSubcoreMesh(
      core_axis_name="c", subcore_axis_name="s", num_cores=NC, num_subcores=NS)

  @functools.partial(
      pl.kernel,
      mesh=mesh,
      out_type=jax.ShapeDtypeStruct((B, D), jnp.float32),
      scratch_types=[
          pltpu.VMEM((2, HC), jnp.int32),
          pltpu.VMEM((2, HC, D), jnp.float32),
          pltpu.VMEM((ACC, D), jnp.float32),
          pltpu.SemaphoreType.DMA,
      ],
  )
  def emb_kernel(x_hbm, table_hbm, out_hbm, idx_v, rows_v, acc_v, sem):
    wid = lax.axis_index("s") * NC + lax.axis_index("c")
    base = wid * B_PER_W

    def chunk_body(c, carry):
      cb = base + c * ACC

      def elem_body(i, carry2):
        bidx = cb + i
        pltpu.sync_copy(x_hbm.at[bidx], idx_v)
        cp0 = pltpu.async_copy(table_hbm.at[idx_v.at[0]], rows_v.at[0], sem)
        cp1 = pltpu.async_copy(table_hbm.at[idx_v.at[1]], rows_v.at[1], sem)
        cp0.wait()
        cp1.wait()

        def red(h, accs):
          return tuple(
              accs[j]
              + rows_v[0, h, pl.ds(16 * j, 16)]
              + rows_v[1, h, pl.ds(16 * j, 16)]
              for j in range(D // 16))

        accs = lax.fori_loop(
            0, HC, red,
            tuple(jnp.zeros((16,), jnp.float32) for _ in range(D // 16)))
        for j in range(D // 16):
          acc_v[i, pl.ds(16 * j, 16)] = accs[j]
        return carry2

      lax.fori_loop(0, ACC, elem_body, 0)
      pltpu.sync_copy(acc_v, out_hbm.at[pl.ds(cb, ACC)])
      return carry

    lax.fori_loop(0, B_PER_W // ACC, chunk_body, 0)

  return emb_kernel(x3, table)


BQ = 1024


def _head_kernel(emb_ref, bias_ref, w_ref, b_ref, out_ref):
  e = emb_ref[...]
  nrm = jnp.sum(e * e, axis=1, keepdims=True)
  q = jnp.maximum(e * lax.rsqrt(nrm) + bias_ref[...], 0.0)
  out_ref[...] = lax.dot_general(
      q, w_ref[...], (((1,), (1,)), ((), ())),
      preferred_element_type=jnp.float32) + b_ref[...]


def _head(emb, bias2, w, b2):
  return pl.pallas_call(
      _head_kernel,
      grid=(B // BQ,),
      in_specs=[
          pl.BlockSpec((BQ, D), lambda i: (i, 0)),
          pl.BlockSpec((1, D), lambda i: (0, 0)),
          pl.BlockSpec((OUT, D), lambda i: (0, 0)),
          pl.BlockSpec((1, OUT), lambda i: (0, 0)),
      ],
      out_specs=pl.BlockSpec((BQ, OUT), lambda i: (i, 0)),
      out_shape=jax.ShapeDtypeStruct((B, OUT), jnp.float32),
  )(emb, bias2, w, b2)


def kernel(x, y, triplet_flag, debug, emb_table, bias, W, b):
  x3 = x.astype(jnp.int32).reshape(B, 2, HC)
  emb = _emb_sum(x3, emb_table)
  return _head(emb, bias.reshape(1, D), W, b.reshape(1, OUT))


# speedup vs baseline: 3.4756x; 1.7578x over previous
import functools

import jax
import jax.numpy as jnp
from jax import lax
from jax.experimental import pallas as pl
from jax.experimental.pallas import tpu as pltpu
from jax.experimental.pallas import tpu_sc as plsc

B = 16384
H = 200
D = 128
OUT = 1000

NC = 2
NS = 16
NW = NC * NS
B_PER_W = B // NW
HC = H // 2
ACC = 32


def _emb_sum(x3, table):
  mesh = plsc.VectorSubcoreMesh(
      core_axis_name="c", subcore_axis_name="s", num_cores=NC, num_subcores=NS)

  @functools.partial(
      pl.kernel,
      mesh=mesh,
      out_type=jax.ShapeDtypeStruct((B, D), jnp.float32),
      scratch_types=[
          pltpu.VMEM((2, 2, HC), jnp.int32),
          pltpu.VMEM((2, 2, HC, D), jnp.float32),
          pltpu.VMEM((ACC, D), jnp.float32),
          pltpu.SemaphoreType.DMA,
          pltpu.SemaphoreType.DMA,
      ],
  )
  def emb_kernel(x_hbm, table_hbm, out_hbm, idx_v, rows_v, acc_v, sem0, sem1):
    wid = lax.axis_index("s") * NC + lax.axis_index("c")
    base = wid * B_PER_W
    sems = (sem0, sem1)

    def start(bidx, s):
      pltpu.sync_copy(x_hbm.at[bidx], idx_v.at[s])
      for j in range(2):
        pltpu.async_copy(
            table_hbm.at[idx_v.at[s].at[j]], rows_v.at[s].at[j], sems[s])

    def drain(s):
      for j in range(2):
        pltpu.make_async_copy(
            table_hbm.at[idx_v.at[s].at[j]], rows_v.at[s].at[j],
            sems[s]).wait()

    def reduce_into(s, row):
      def red(h, accs):
        return tuple(
            accs[j]
            + rows_v[s, 0, h, pl.ds(16 * j, 16)]
            + rows_v[s, 1, h, pl.ds(16 * j, 16)]
            for j in range(D // 16))

      accs = lax.fori_loop(
          0, HC, red,
          tuple(jnp.zeros((16,), jnp.float32) for _ in range(D // 16)))
      for j in range(D // 16):
        acc_v[row, pl.ds(16 * j, 16)] = accs[j]

    start(base, 0)
    start(base + 1, 1)

    def pair_body(g2, carry):
      e0 = 2 * g2
      for s in range(2):
        e = e0 + s
        drain(s)
        reduce_into(s, lax.rem(e, ACC))

        @pl.when(e + 2 < B_PER_W)
        def _():
          start(base + e + 2, s)

      @pl.when(lax.rem(e0 + 1, ACC) == ACC - 1)
      def _():
        off = pl.multiple_of(base + e0 + 2 - ACC, ACC)
        pltpu.sync_copy(acc_v, out_hbm.at[pl.ds(off, ACC)])

      return carry

    lax.fori_loop(0, B_PER_W // 2, pair_body, 0)

  return emb_kernel(x3, table)


BQ = 1024


def _head_kernel(emb_ref, bias_ref, w_ref, b_ref, out_ref):
  e = emb_ref[...]
  nrm = jnp.sum(e * e, axis=1, keepdims=True)
  q = jnp.maximum(e * lax.rsqrt(nrm) + bias_ref[...], 0.0)
  out_ref[...] = lax.dot_general(
      q, w_ref[...], (((1,), (1,)), ((), ())),
      preferred_element_type=jnp.float32) + b_ref[...]


def _head(emb, bias2, w, b2):
  return pl.pallas_call(
      _head_kernel,
      grid=(B // BQ,),
      in_specs=[
          pl.BlockSpec((BQ, D), lambda i: (i, 0)),
          pl.BlockSpec((1, D), lambda i: (0, 0)),
          pl.BlockSpec((OUT, D), lambda i: (0, 0)),
          pl.BlockSpec((1, OUT), lambda i: (0, 0)),
      ],
      out_specs=pl.BlockSpec((BQ, OUT), lambda i: (i, 0)),
      out_shape=jax.ShapeDtypeStruct((B, OUT), jnp.float32),
  )(emb, bias2, w, b2)


def kernel(x, y, triplet_flag, debug, emb_table, bias, W, b):
  x3 = x.astype(jnp.int32).reshape(B, 2, HC)
  emb = _emb_sum(x3, emb_table)
  return _head(emb, bias.reshape(1, D), W, b.reshape(1, OUT))


# speedup vs baseline: 4.0241x; 1.1578x over previous
import functools

import jax
import jax.numpy as jnp
from jax import lax
from jax.experimental import pallas as pl
from jax.experimental.pallas import tpu as pltpu
from jax.experimental.pallas import tpu_sc as plsc

B = 16384
H = 200
D = 128
OUT = 1000

NC = 2
NS = 16
NW = NC * NS
B_PER_W = B // NW
HC = H // 2
ACC = 32
IDXB = 32


def _emb_sum(x3, table):
  mesh = plsc.VectorSubcoreMesh(
      core_axis_name="c", subcore_axis_name="s", num_cores=NC, num_subcores=NS)

  nblk = B_PER_W // IDXB

  @functools.partial(
      pl.kernel,
      mesh=mesh,
      out_type=jax.ShapeDtypeStruct((B, D), jnp.float32),
      scratch_types=[
          pltpu.VMEM((2, IDXB, 2, HC), jnp.int32),
          pltpu.VMEM((2, 2, HC, D), jnp.float32),
          pltpu.VMEM((ACC, D), jnp.float32),
          pltpu.SemaphoreType.DMA,
          pltpu.SemaphoreType.DMA,
          pltpu.SemaphoreType.DMA,
      ],
  )
  def emb_kernel(x_hbm, table_hbm, out_hbm, idxblk_v, rows_v, acc_v,
                 sem0, sem1, isem):
    wid = lax.axis_index("s") * NC + lax.axis_index("c")
    base = wid * B_PER_W
    sems = (sem0, sem1)

    def idx_copy(blk, slot):
      return pltpu.make_async_copy(
          x_hbm.at[pl.ds(base + blk * IDXB, IDXB)], idxblk_v.at[slot], isem)

    def fire(e, s):
      blk_slot = lax.rem(e // IDXB, 2)
      r = lax.rem(e, IDXB)
      for j in range(2):
        pltpu.async_copy(
            table_hbm.at[idxblk_v.at[blk_slot].at[r].at[j]],
            rows_v.at[s].at[j], sems[s])

    def drain(s):
      for j in range(2):
        pltpu.make_async_copy(
            table_hbm.at[idxblk_v.at[0].at[0].at[j]], rows_v.at[s].at[j],
            sems[s]).wait()

    def reduce_into(s, row):
      def red(h, accs):
        return tuple(
            accs[j]
            + rows_v[s, 0, h, pl.ds(16 * j, 16)]
            + rows_v[s, 1, h, pl.ds(16 * j, 16)]
            for j in range(D // 16))

      accs = lax.fori_loop(
          0, HC, red,
          tuple(jnp.zeros((16,), jnp.float32) for _ in range(D // 16)),
          unroll=2)
      for j in range(D // 16):
        acc_v[row, pl.ds(16 * j, 16)] = accs[j]

    idx_copy(0, 0).start()
    idx_copy(0, 0).wait()
    idx_copy(1, 1).start()
    fire(0, 0)
    fire(1, 1)

    def pair_body(g2, carry):
      e0 = 2 * g2
      for s in range(2):
        e = e0 + s
        drain(s)
        reduce_into(s, lax.rem(e, ACC))

        nxt = e + 2
        rn = lax.rem(nxt, IDXB)

        @pl.when(nxt < B_PER_W)
        def _():
          @pl.when(rn == 0)
          def _():
            idx_copy(0, 0).wait()

          fire(nxt, s)

          @pl.when(jnp.logical_and(rn == 4, nxt < (nblk - 1) * IDXB))
          def _():
            nxtblk = nxt // IDXB + 1
            idx_copy(nxtblk, lax.rem(nxtblk, 2)).start()

      @pl.when(lax.rem(e0 + 1, ACC) == ACC - 1)
      def _():
        off = pl.multiple_of(base + e0 + 2 - ACC, ACC)
        pltpu.sync_copy(acc_v, out_hbm.at[pl.ds(off, ACC)])

      return carry

    lax.fori_loop(0, B_PER_W // 2, pair_body, 0)

  return emb_kernel(x3, table)


BQ = 1024


def _head_kernel(emb_ref, bias_ref, w_ref, b_ref, out_ref):
  e = emb_ref[...]
  nrm = jnp.sum(e * e, axis=1, keepdims=True)
  q = jnp.maximum(e * lax.rsqrt(nrm) + bias_ref[...], 0.0)
  out_ref[...] = lax.dot_general(
      q, w_ref[...], (((1,), (1,)), ((), ())),
      preferred_element_type=jnp.float32) + b_ref[...]


def _head(emb, bias2, w, b2):
  return pl.pallas_call(
      _head_kernel,
      grid=(B // BQ,),
      in_specs=[
          pl.BlockSpec((BQ, D), lambda i: (i, 0)),
          pl.BlockSpec((1, D), lambda i: (0, 0)),
          pl.BlockSpec((OUT, D), lambda i: (0, 0)),
          pl.BlockSpec((1, OUT), lambda i: (0, 0)),
      ],
      out_specs=pl.BlockSpec((BQ, OUT), lambda i: (i, 0)),
      out_shape=jax.ShapeDtypeStruct((B, OUT), jnp.float32),
  )(emb, bias2, w, b2)


def kernel(x, y, triplet_flag, debug, emb_table, bias, W, b):
  x3 = x.astype(jnp.int32).reshape(B, 2, HC)
  emb = _emb_sum(x3, emb_table)
  return _head(emb, bias.reshape(1, D), W, b.reshape(1, OUT))


# speedup vs baseline: 4.0359x; 1.0029x over previous
import functools

import jax
import jax.numpy as jnp
from jax import lax
from jax.experimental import pallas as pl
from jax.experimental.pallas import tpu as pltpu
from jax.experimental.pallas import tpu_sc as plsc

B = 16384
H = 200
D = 128
OUT = 1000

NC = 2
NS = 16
NW = NC * NS
B_PER_W = B // NW
HC = H // 2
ACC = 32
IDXB = 32
NSLOT = 2


def _emb_sum(x3, table):
  mesh = plsc.VectorSubcoreMesh(
      core_axis_name="c", subcore_axis_name="s", num_cores=NC, num_subcores=NS)

  nblk = B_PER_W // IDXB

  @functools.partial(
      pl.kernel,
      mesh=mesh,
      out_type=jax.ShapeDtypeStruct((B, D), jnp.float32),
      scratch_types=[
          pltpu.VMEM((2, IDXB, 2, HC), jnp.int32),
          pltpu.VMEM((NSLOT, 2, HC, D), jnp.float32),
          pltpu.VMEM((ACC, D), jnp.float32),
          pltpu.SemaphoreType.DMA,
          pltpu.SemaphoreType.DMA,
          pltpu.SemaphoreType.DMA,
      ],
  )
  def emb_kernel(x_hbm, table_hbm, out_hbm, idxblk_v, rows_v, acc_v,
                 sem0, sem1, isem):
    wid = lax.axis_index("s") * NC + lax.axis_index("c")
    base = wid * B_PER_W
    sems = (sem0, sem1)

    def idx_copy(blk, slot):
      return pltpu.make_async_copy(
          x_hbm.at[pl.ds(base + blk * IDXB, IDXB)], idxblk_v.at[slot], isem)

    def fire(e, s):
      blk_slot = lax.rem(e // IDXB, 2)
      r = lax.rem(e, IDXB)
      for j in range(2):
        pltpu.async_copy(
            table_hbm.at[idxblk_v.at[blk_slot].at[r].at[j]],
            rows_v.at[s].at[j], sems[s])

    def drain(s):
      for j in range(2):
        pltpu.make_async_copy(
            table_hbm.at[idxblk_v.at[0].at[0].at[j]], rows_v.at[s].at[j],
            sems[s]).wait()

    def reduce_into(s, row):
      def red(h, accs):
        return tuple(
            accs[j]
            + rows_v[s, 0, h, pl.ds(16 * j, 16)]
            + rows_v[s, 1, h, pl.ds(16 * j, 16)]
            for j in range(D // 16))

      accs = lax.fori_loop(
          0, HC, red,
          tuple(jnp.zeros((16,), jnp.float32) for _ in range(D // 16)),
          unroll=4)
      for j in range(D // 16):
        acc_v[row, pl.ds(16 * j, 16)] = accs[j]

    idx_copy(0, 0).start()
    idx_copy(0, 0).wait()
    idx_copy(1, 1).start()
    for s in range(NSLOT):
      fire(s, s)

    def grp_body(gg, carry):
      e0 = NSLOT * gg
      for s in range(NSLOT):
        e = e0 + s
        drain(s)
        reduce_into(s, lax.rem(e, ACC))

        nxt = e + NSLOT
        rn = lax.rem(nxt, IDXB)

        @pl.when(nxt < B_PER_W)
        def _():
          @pl.when(rn == 0)
          def _():
            idx_copy(0, 0).wait()

          fire(nxt, s)

          @pl.when(jnp.logical_and(rn == NSLOT, nxt < (nblk - 1) * IDXB))
          def _():
            nxtblk = nxt // IDXB + 1
            idx_copy(nxtblk, lax.rem(nxtblk, 2)).start()

      @pl.when(lax.rem(e0 + NSLOT - 1, ACC) == ACC - 1)
      def _():
        off = pl.multiple_of(base + e0 + NSLOT - ACC, ACC)
        pltpu.sync_copy(acc_v, out_hbm.at[pl.ds(off, ACC)])

      return carry

    lax.fori_loop(0, B_PER_W // NSLOT, grp_body, 0)

  return emb_kernel(x3, table)


BQ = 1024


def _head_kernel(emb_ref, bias_ref, w_ref, b_ref, out_ref):
  e = emb_ref[...]
  nrm = jnp.sum(e * e, axis=1, keepdims=True)
  q = jnp.maximum(e * lax.rsqrt(nrm) + bias_ref[...], 0.0)
  out_ref[...] = lax.dot_general(
      q, w_ref[...], (((1,), (1,)), ((), ())),
      preferred_element_type=jnp.float32) + b_ref[...]


def _head(emb, bias2, w, b2):
  return pl.pallas_call(
      _head_kernel,
      grid=(B // BQ,),
      in_specs=[
          pl.BlockSpec((BQ, D), lambda i: (i, 0)),
          pl.BlockSpec((1, D), lambda i: (0, 0)),
          pl.BlockSpec((OUT, D), lambda i: (0, 0)),
          pl.BlockSpec((1, OUT), lambda i: (0, 0)),
      ],
      out_specs=pl.BlockSpec((BQ, OUT), lambda i: (i, 0)),
      out_shape=jax.ShapeDtypeStruct((B, OUT), jnp.float32),
  )(emb, bias2, w, b2)


def kernel(x, y, triplet_flag, debug, emb_table, bias, W, b):
  x3 = x.astype(jnp.int32).reshape(B, 2, HC)
  emb = _emb_sum(x3, emb_table)
  return _head(emb, bias.reshape(1, D), W, b.reshape(1, OUT))


# speedup vs baseline: 4.0366x; 1.0002x over previous
import functools

import jax
import jax.numpy as jnp
from jax import lax
from jax.experimental import pallas as pl
from jax.experimental.pallas import tpu as pltpu
from jax.experimental.pallas import tpu_sc as plsc

B = 16384
H = 200
D = 128
OUT = 1000

NC = 2
NS = 16
NW = NC * NS
HC = H // 2
ACC = 32
IDXB = 32
NSLOT = 2
NCHUNK = 1
CB = B // NCHUNK
BQ = 1024


def _emb_sum(x, table, dep, row0, nrows):
  b_per_w = nrows // NW
  nblk = b_per_w // IDXB
  mesh = plsc.VectorSubcoreMesh(
      core_axis_name="c", subcore_axis_name="s", num_cores=NC, num_subcores=NS)

  @functools.partial(
      pl.kernel,
      mesh=mesh,
      out_type=jax.ShapeDtypeStruct((nrows, D), jnp.float32),
      scratch_types=[
          pltpu.VMEM((2, IDXB, 2, HC), jnp.int32),
          pltpu.VMEM((NSLOT, 2, HC, D), jnp.float32),
          pltpu.VMEM((ACC, D), jnp.float32),
          pltpu.SemaphoreType.DMA,
          pltpu.SemaphoreType.DMA,
          pltpu.SemaphoreType.DMA,
      ],
  )
  def emb_kernel(x_hbm, table_hbm, dep_hbm, out_hbm, idxblk_v, rows_v, acc_v,
                 sem0, sem1, isem):
    del dep_hbm
    wid = lax.axis_index("s") * NC + lax.axis_index("c")
    base = wid * b_per_w
    xbase = row0 + base
    sems = (sem0, sem1)

    def idx_copy(blk, slot):
      return pltpu.make_async_copy(
          x_hbm.at[pl.ds(xbase + blk * IDXB, IDXB)], idxblk_v.at[slot], isem)

    def fire(e, s):
      blk_slot = lax.rem(e // IDXB, 2)
      r = lax.rem(e, IDXB)
      for j in range(2):
        for off, ln in ((0, 48), (48, 52)):
          pltpu.async_copy(
              table_hbm.at[idxblk_v.at[blk_slot].at[r].at[j]
                           .at[pl.ds(off, ln)]],
              rows_v.at[s].at[j].at[pl.ds(off, ln)], sems[s])

    def drain(s):
      for j in range(2):
        for off, ln in ((0, 48), (48, 52)):
          pltpu.make_async_copy(
              table_hbm.at[idxblk_v.at[0].at[0].at[j].at[pl.ds(off, ln)]],
              rows_v.at[s].at[j].at[pl.ds(off, ln)],
              sems[s]).wait()

    def reduce_into(s, row):
      def red(h, accs):
        return tuple(
            accs[j]
            + rows_v[s, 0, h, pl.ds(16 * j, 16)]
            + rows_v[s, 1, h, pl.ds(16 * j, 16)]
            for j in range(D // 16))

      accs = lax.fori_loop(
          0, HC, red,
          tuple(jnp.zeros((16,), jnp.float32) for _ in range(D // 16)),
          unroll=4)
      for j in range(D // 16):
        acc_v[row, pl.ds(16 * j, 16)] = accs[j]

    idx_copy(0, 0).start()
    idx_copy(0, 0).wait()
    idx_copy(1, 1).start()
    for s in range(NSLOT):
      fire(s, s)

    def grp_body(gg, carry):
      e0 = NSLOT * gg
      for s in range(NSLOT):
        e = e0 + s
        drain(s)
        reduce_into(s, lax.rem(e, ACC))

        nxt = e + NSLOT
        rn = lax.rem(nxt, IDXB)

        @pl.when(nxt < b_per_w)
        def _():
          @pl.when(rn == 0)
          def _():
            idx_copy(0, 0).wait()

          fire(nxt, s)

          @pl.when(jnp.logical_and(rn == NSLOT, nxt < (nblk - 1) * IDXB))
          def _():
            nxtblk = nxt // IDXB + 1
            idx_copy(nxtblk, lax.rem(nxtblk, 2)).start()

      @pl.when(lax.rem(e0 + NSLOT - 1, ACC) == ACC - 1)
      def _():
        off = pl.multiple_of(base + e0 + NSLOT - ACC, ACC)
        pltpu.sync_copy(acc_v, out_hbm.at[pl.ds(off, ACC)])

      return carry

    lax.fori_loop(0, b_per_w // NSLOT, grp_body, 0)

  return emb_kernel(x, table, dep)


def _head_math(emb_ref, bias_ref, w_ref, b_ref, out_ref):
  e = emb_ref[...]
  nrm = jnp.sum(e * e, axis=1, keepdims=True)
  q = jnp.maximum(e * lax.rsqrt(nrm) + bias_ref[...], 0.0)
  out_ref[...] = lax.dot_general(
      q, w_ref[...], (((1,), (1,)), ((), ())),
      preferred_element_type=jnp.float32) + b_ref[...]


def _head_first(emb_ref, bias_ref, w_ref, b_ref, out_ref):
  _head_math(emb_ref, bias_ref, w_ref, b_ref, out_ref)


def _head_next(emb_ref, bias_ref, w_ref, b_ref, prev_ref, out_ref):
  del prev_ref
  _head_math(emb_ref, bias_ref, w_ref, b_ref, out_ref)


def _head_chunk(emb_c, bias2, w, b2, prev, blk_off):
  common_specs = [
      pl.BlockSpec((BQ, D), lambda i: (i, 0)),
      pl.BlockSpec((1, D), lambda i: (0, 0)),
      pl.BlockSpec((OUT, D), lambda i: (0, 0)),
      pl.BlockSpec((1, OUT), lambda i: (0, 0)),
  ]
  out_spec = pl.BlockSpec((BQ, OUT), lambda i, o=blk_off: (i + o, 0))
  if prev is None:
    return pl.pallas_call(
        _head_first,
        grid=(CB // BQ,),
        in_specs=common_specs,
        out_specs=out_spec,
        out_shape=jax.ShapeDtypeStruct((B, OUT), jnp.float32),
    )(emb_c, bias2, w, b2)
  return pl.pallas_call(
      _head_next,
      grid=(CB // BQ,),
      in_specs=common_specs + [pl.BlockSpec((8, 128), lambda i: (0, 0))],
      out_specs=out_spec,
      out_shape=jax.ShapeDtypeStruct((B, OUT), jnp.float32),
      input_output_aliases={4: 0},
  )(emb_c, bias2, w, b2, prev)


def kernel(x, y, triplet_flag, debug, emb_table, bias, W, b):
  xi = x.astype(jnp.int32).reshape(B, 2, HC)
  bias2 = bias.reshape(1, D)
  b2 = b.reshape(1, OUT)
  embs = []
  dep = bias2
  for k in range(NCHUNK):
    e_k = _emb_sum(xi, emb_table, dep, k * CB, CB)
    embs.append(e_k)
    dep = e_k
  emb = jnp.concatenate(embs, axis=0)
  return pl.pallas_call(
      _head_first,
      grid=(B // BQ,),
      in_specs=[
          pl.BlockSpec((BQ, D), lambda i: (i, 0)),
          pl.BlockSpec((1, D), lambda i: (0, 0)),
          pl.BlockSpec((OUT, D), lambda i: (0, 0)),
          pl.BlockSpec((1, OUT), lambda i: (0, 0)),
      ],
      out_specs=pl.BlockSpec((BQ, OUT), lambda i: (i, 0)),
      out_shape=jax.ShapeDtypeStruct((B, OUT), jnp.float32),
  )(emb, bias2, W, b2)
